# baseline (speedup 1.0000x reference)
import jax
import jax.numpy as jnp
from jax.experimental import pallas as pl
from jax.experimental.pallas import tpu as pltpu

_WARMUP = (80, 80, 240)
_BM = 400
_NBUF = 3
_NOBUF = 4


def _block_schedule(n):
    sizes = list(_WARMUP)
    assert sum(_WARMUP) == _BM and n % _BM == 0
    sizes += [_BM] * (n // _BM - 1)
    offs = [0]
    for s in sizes[:-1]:
        offs.append(offs[-1] + s)
    return list(zip(offs, sizes))


def _hbs_kernel(x_ref, w_ref, nb_hbm, o_hbm, m_ref, nb_buf, out_buf,
                sem_in, sem_out):
    n = x_ref.shape[0]
    sched = _block_schedule(n)
    nblocks = len(sched)

    def copy_in(block, slot):
        off, sz = sched[block]
        return pltpu.make_async_copy(
            nb_hbm.at[pl.ds(off, sz), :],
            nb_buf.at[slot, pl.ds(0, sz), :],
            sem_in.at[slot],
        )

    def copy_out(block, slot):
        off, sz = sched[block]
        return pltpu.make_async_copy(
            out_buf.at[slot, pl.ds(0, sz), :],
            o_hbm.at[pl.ds(off, sz), :],
            sem_out.at[slot],
        )

    for j in range(_NBUF):
        copy_in(j, j).start()

    m_ref[...] = jnp.dot(
        x_ref[...], w_ref[...], preferred_element_type=jnp.float32
    )

    for i in range(nblocks):
        slot = i % _NBUF
        oslot = i % _NOBUF
        _, sz = sched[i]
        copy_in(i, slot).wait()
        if i >= _NOBUF:
            copy_out(i - _NOBUF, oslot).wait()
        acc = jnp.dot(nb_buf[slot, :sz, :], m_ref[...],
                      preferred_element_type=jnp.float32)
        out_buf[oslot, :sz, :] = jnp.maximum(acc, 0.0)
        copy_out(i, oslot).start()
        nxt = i + _NBUF
        if nxt < nblocks:
            copy_in(nxt, slot).start()

    for j in range(_NOBUF):
        blk = nblocks - _NOBUF + j
        copy_out(blk, blk % _NOBUF).wait()


def kernel(x_source, neighborhood, weight, weight2, weight3):
    n, d_in = x_source.shape
    d_out = weight.shape[1]

    out = pl.pallas_call(
        _hbs_kernel,
        in_specs=[
            pl.BlockSpec((n, d_in), lambda: (0, 0)),
            pl.BlockSpec((d_in, d_out), lambda: (0, 0)),
            pl.BlockSpec(memory_space=pl.ANY),
        ],
        out_specs=pl.BlockSpec(memory_space=pl.ANY),
        out_shape=jax.ShapeDtypeStruct((n, d_out), jnp.float32),
        scratch_shapes=[
            pltpu.VMEM((n, d_out), jnp.float32),
            pltpu.VMEM((_NBUF, _BM, n), jnp.float32),
            pltpu.VMEM((_NOBUF, _BM, d_out), jnp.float32),
            pltpu.SemaphoreType.DMA((_NBUF,)),
            pltpu.SemaphoreType.DMA((_NOBUF,)),
        ],
        compiler_params=pltpu.CompilerParams(
            vmem_limit_bytes=67108864,
        ),
    )(x_source, weight, neighborhood)
    return out


# speedup vs baseline: 1.0024x; 1.0024x over previous
import jax
import jax.numpy as jnp
from jax.experimental import pallas as pl
from jax.experimental.pallas import tpu as pltpu


def _fused_kernel(x_ref, w_ref, nb_ref, o_ref, m_ref):
    @pl.when(pl.program_id(0) == 0)
    def _():
        m_ref[...] = jnp.dot(
            x_ref[...], w_ref[...], preferred_element_type=jnp.float32
        )

    acc = jnp.dot(nb_ref[...], m_ref[...],
                  preferred_element_type=jnp.float32)
    o_ref[...] = jnp.maximum(acc, 0.0)


def kernel(x_source, neighborhood, weight, weight2, weight3):
    n, d_in = x_source.shape
    d_out = weight.shape[1]

    bm = 640
    out = pl.pallas_call(
        _fused_kernel,
        grid=(pl.cdiv(n, bm),),
        in_specs=[
            pl.BlockSpec((n, d_in), lambda i: (0, 0)),
            pl.BlockSpec((d_in, d_out), lambda i: (0, 0)),
            pl.BlockSpec((bm, n), lambda i: (i, 0)),
        ],
        out_specs=pl.BlockSpec((bm, d_out), lambda i: (i, 0)),
        out_shape=jax.ShapeDtypeStruct((n, d_out), jnp.float32),
        scratch_shapes=[pltpu.VMEM((n, d_out), jnp.float32)],
        compiler_params=pltpu.CompilerParams(
            dimension_semantics=("arbitrary",),
            vmem_limit_bytes=67108864,
        ),
    )(x_source, weight, neighborhood)
    return out


# speedup vs baseline: 1.0238x; 1.0214x over previous
import jax
import jax.numpy as jnp
from jax.experimental import pallas as pl
from jax.experimental.pallas import tpu as pltpu


def _fused_kernel(x_ref, w_ref, nb_ref, o_ref, m_ref):
    @pl.when(pl.program_id(0) == 0)
    def _():
        m_ref[...] = jnp.dot(
            x_ref[...], w_ref[...], preferred_element_type=jnp.float32
        )

    acc = jnp.dot(nb_ref[...], m_ref[...],
                  preferred_element_type=jnp.float32)
    o_ref[...] = jnp.maximum(acc, 0.0)


def kernel(x_source, neighborhood, weight, weight2, weight3):
    n, d_in = x_source.shape
    d_out = weight.shape[1]

    bm = 400
    out = pl.pallas_call(
        _fused_kernel,
        grid=(n // bm,),
        in_specs=[
            pl.BlockSpec((n, d_in), lambda i: (0, 0)),
            pl.BlockSpec((d_in, d_out), lambda i: (0, 0)),
            pl.BlockSpec((bm, n), lambda i: (i, 0)),
        ],
        out_specs=pl.BlockSpec((bm, d_out), lambda i: (i, 0)),
        out_shape=jax.ShapeDtypeStruct((n, d_out), jnp.float32),
        scratch_shapes=[pltpu.VMEM((n, d_out), jnp.float32)],
        compiler_params=pltpu.CompilerParams(
            dimension_semantics=("arbitrary",),
        ),
    )(x_source, weight, neighborhood)
    return out
